# baseline (speedup 1.0000x reference)
import functools
import math

import numpy as np
import jax
import jax.numpy as jnp
from jax import lax
from jax.experimental import pallas as pl
from jax.experimental.pallas import tpu as pltpu
from jax.experimental.pallas import tpu_sc as plsc

VOCAB = 1000000
EMBED_DIM = 64
SEQ = 512
LANES = 16


def _make_pe(max_seq_len, embed_dim):
    position = np.arange(0, max_seq_len, dtype=np.float32)[:, None]
    div_term = np.exp(
        np.arange(0, embed_dim, 2, dtype=np.float32) * (-math.log(10000.0) / embed_dim)
    )
    pe = np.zeros((max_seq_len, embed_dim), dtype=np.float32)
    pe[:, 0::2] = np.sin(position * div_term)
    pe[:, 1::2] = np.cos(position * div_term)
    return pe


_PE = _make_pe(SEQ, EMBED_DIM)


def kernel(token_ids, token_embed_weight):
    ids = token_ids.astype(jnp.int32)
    pe = jnp.asarray(_PE)

    info = plsc.get_sparse_core_info()
    nc, ns = info.num_cores, info.num_subcores
    nw = nc * ns
    bpw = SEQ // nw

    mesh = plsc.VectorSubcoreMesh(core_axis_name="c", subcore_axis_name="s")

    @functools.partial(
        pl.kernel,
        mesh=mesh,
        out_type=jax.ShapeDtypeStruct((SEQ, EMBED_DIM), jnp.float32),
        scratch_types=[
            pltpu.VMEM((bpw,), jnp.int32),
            pltpu.VMEM((bpw, EMBED_DIM), jnp.float32),
            pltpu.VMEM((bpw, EMBED_DIM), jnp.float32),
            pltpu.SemaphoreType.DMA,
        ],
        compiler_params=pltpu.CompilerParams(use_tc_tiling_on_sc=False),
    )
    def emb_kernel(ids_hbm, table_hbm, pe_hbm, out_hbm, idx_v, rows_v, pe_v, sem):
        wid = lax.axis_index("s") * nc + lax.axis_index("c")
        base = wid * bpw
        pltpu.sync_copy(ids_hbm.at[pl.ds(base, bpw)], idx_v)
        gather = pltpu.async_copy(table_hbm.at[idx_v], rows_v, sem)
        pltpu.sync_copy(pe_hbm.at[pl.ds(base, bpw)], pe_v)
        gather.wait()
        for i in range(bpw):
            for j in range(EMBED_DIM // LANES):
                sl = pl.ds(j * LANES, LANES)
                rows_v[i, sl] = rows_v[i, sl] + pe_v[i, sl]
        pltpu.sync_copy(rows_v, out_hbm.at[pl.ds(base, bpw)])

    return emb_kernel(ids, token_embed_weight, pe)


# speedup vs baseline: 1.7233x; 1.7233x over previous
import functools
import math

import numpy as np
import jax
import jax.numpy as jnp
from jax import lax
from jax.experimental import pallas as pl
from jax.experimental.pallas import tpu as pltpu
from jax.experimental.pallas import tpu_sc as plsc

VOCAB = 1000000
EMBED_DIM = 64
SEQ = 512
LANES = 16


def _make_pe(max_seq_len, embed_dim):
    position = np.arange(0, max_seq_len, dtype=np.float32)[:, None]
    div_term = np.exp(
        np.arange(0, embed_dim, 2, dtype=np.float32) * (-math.log(10000.0) / embed_dim)
    )
    pe = np.zeros((max_seq_len, embed_dim), dtype=np.float32)
    pe[:, 0::2] = np.sin(position * div_term)
    pe[:, 1::2] = np.cos(position * div_term)
    return pe


_PE = _make_pe(SEQ, EMBED_DIM)


def kernel(token_ids, token_embed_weight):
    ids = token_ids.astype(jnp.int32)
    pe = jnp.asarray(_PE)

    info = plsc.get_sparse_core_info()
    nc, ns = info.num_cores, info.num_subcores
    nw = nc * ns
    bpw = SEQ // nw

    mesh = plsc.VectorSubcoreMesh(core_axis_name="c", subcore_axis_name="s")

    @functools.partial(
        pl.kernel,
        mesh=mesh,
        out_type=jax.ShapeDtypeStruct((SEQ, EMBED_DIM), jnp.float32),
        scratch_types=[
            pltpu.VMEM((bpw,), jnp.int32),
            pltpu.VMEM((bpw, EMBED_DIM), jnp.float32),
            pltpu.VMEM((bpw, EMBED_DIM), jnp.float32),
            pltpu.SemaphoreType.DMA,
        ],
    )
    def emb_kernel(ids_hbm, table_hbm, pe_hbm, out_hbm, ids_v, rows_v, pe_v, sem):
        wid = lax.axis_index("s") * nc + lax.axis_index("c")
        base = wid * bpw
        pltpu.sync_copy(ids_hbm.at[pl.ds(base, bpw)], ids_v)
        idvec = ids_v[...]
        copies = []
        for i in range(bpw):
            tid = idvec[i]
            copies.append(
                pltpu.async_copy(
                    table_hbm.at[pl.ds(tid, 1)], rows_v.at[pl.ds(i, 1)], sem
                )
            )
        pltpu.sync_copy(pe_hbm.at[pl.ds(base, bpw)], pe_v)
        for cp in copies:
            cp.wait()
        for i in range(bpw):
            for j in range(EMBED_DIM // LANES):
                sl = pl.ds(j * LANES, LANES)
                rows_v[i, sl] = rows_v[i, sl] + pe_v[i, sl]
        pltpu.sync_copy(rows_v, out_hbm.at[pl.ds(base, bpw)])

    return emb_kernel(ids, token_embed_weight, pe)


# speedup vs baseline: 21.6975x; 12.5908x over previous
import functools
import math

import numpy as np
import jax
import jax.numpy as jnp
from jax import lax
from jax.experimental import pallas as pl
from jax.experimental.pallas import tpu as pltpu
from jax.experimental.pallas import tpu_sc as plsc

VOCAB = 1000000
EMBED_DIM = 64
SEQ = 512
LANES = 16
BLK = 128
NBUF = 8


def _make_pe(max_seq_len, embed_dim):
    position = np.arange(0, max_seq_len, dtype=np.float32)[:, None]
    div_term = np.exp(
        np.arange(0, embed_dim, 2, dtype=np.float32) * (-math.log(10000.0) / embed_dim)
    )
    pe = np.zeros((max_seq_len, embed_dim), dtype=np.float32)
    pe[:, 0::2] = np.sin(position * div_term)
    pe[:, 1::2] = np.cos(position * div_term)
    return pe


_PE = _make_pe(SEQ, EMBED_DIM)


def kernel(token_ids, token_embed_weight):
    ids = token_ids.astype(jnp.int32)
    pe = jnp.asarray(_PE)
    table_t = token_embed_weight.T

    info = plsc.get_sparse_core_info()
    nc, ns = info.num_cores, info.num_subcores
    nw = nc * ns
    bpw = SEQ // nw

    mesh = plsc.VectorSubcoreMesh(core_axis_name="c", subcore_axis_name="s")

    @functools.partial(
        pl.kernel,
        mesh=mesh,
        out_type=jax.ShapeDtypeStruct((SEQ, EMBED_DIM), jnp.float32),
        scratch_types=[
            pltpu.VMEM((bpw,), jnp.int32),
            pltpu.VMEM((bpw, EMBED_DIM), jnp.float32),
            pltpu.VMEM((bpw, EMBED_DIM), jnp.float32),
            [pltpu.VMEM((EMBED_DIM, BLK), jnp.float32) for _ in range(NBUF)],
            pltpu.SemaphoreType.DMA,
        ],
        compiler_params=pltpu.CompilerParams(needs_layout_passes=False),
    )
    def emb_kernel(ids_hbm, table_hbm, pe_hbm, out_hbm, ids_v, pe_v, out_v, bufs, sem):
        wid = lax.axis_index("s") * nc + lax.axis_index("c")
        base = wid * bpw
        pltpu.sync_copy(ids_hbm.at[pl.ds(base, bpw)], ids_v)
        idvec = ids_v[...]
        blocks = (idvec >> 7) << 7
        lanes = idvec & (BLK - 1)
        iota = lax.iota(jnp.int32, LANES)

        def fire(i):
            blk = pl.multiple_of(blocks[i], BLK)
            return pltpu.async_copy(
                table_hbm.at[:, pl.ds(blk, BLK)], bufs[i % NBUF], sem
            )

        cps = [None] * bpw
        for i in range(NBUF):
            cps[i] = fire(i)
        pltpu.sync_copy(pe_hbm.at[pl.ds(base, bpw)], pe_v)
        for i in range(bpw):
            cps[i].wait()
            buf = bufs[i % NBUF]
            lane_b = jnp.broadcast_to(lanes[i], (LANES,))
            for j in range(EMBED_DIM // LANES):
                vals = plsc.load_gather(buf, [iota + (j * LANES), lane_b])
                sl = pl.ds(j * LANES, LANES)
                out_v[i, sl] = vals + pe_v[i, sl]
            if i + NBUF < bpw:
                cps[i + NBUF] = fire(i + NBUF)
        pltpu.sync_copy(out_v, out_hbm.at[pl.ds(base, bpw)])

    return emb_kernel(ids, table_t, pe)


# speedup vs baseline: 21.7549x; 1.0026x over previous
import functools
import math

import numpy as np
import jax
import jax.numpy as jnp
from jax import lax
from jax.experimental import pallas as pl
from jax.experimental.pallas import tpu as pltpu
from jax.experimental.pallas import tpu_sc as plsc

VOCAB = 1000000
EMBED_DIM = 64
SEQ = 512
LANES = 16
BLK = 128
NBUF = 14


def _make_pe(max_seq_len, embed_dim):
    position = np.arange(0, max_seq_len, dtype=np.float32)[:, None]
    div_term = np.exp(
        np.arange(0, embed_dim, 2, dtype=np.float32) * (-math.log(10000.0) / embed_dim)
    )
    pe = np.zeros((max_seq_len, embed_dim), dtype=np.float32)
    pe[:, 0::2] = np.sin(position * div_term)
    pe[:, 1::2] = np.cos(position * div_term)
    return pe


_PE = _make_pe(SEQ, EMBED_DIM)


def kernel(token_ids, token_embed_weight):
    ids = token_ids.astype(jnp.int32)
    pe = jnp.asarray(_PE)
    table_t = token_embed_weight.T

    info = plsc.get_sparse_core_info()
    nc, ns = info.num_cores, info.num_subcores
    nw = nc * ns
    bpw = SEQ // nw

    mesh = plsc.VectorSubcoreMesh(core_axis_name="c", subcore_axis_name="s")

    @functools.partial(
        pl.kernel,
        mesh=mesh,
        out_type=jax.ShapeDtypeStruct((SEQ, EMBED_DIM), jnp.float32),
        scratch_types=[
            pltpu.VMEM((bpw,), jnp.int32),
            pltpu.VMEM((bpw, EMBED_DIM), jnp.float32),
            pltpu.VMEM((bpw, EMBED_DIM), jnp.float32),
            [pltpu.VMEM((EMBED_DIM, BLK), jnp.float32) for _ in range(NBUF)],
            pltpu.SemaphoreType.DMA,
        ],
        compiler_params=pltpu.CompilerParams(needs_layout_passes=False),
    )
    def emb_kernel(ids_hbm, table_hbm, pe_hbm, out_hbm, ids_v, pe_v, out_v, bufs, sem):
        wid = lax.axis_index("s") * nc + lax.axis_index("c")
        base = wid * bpw
        pltpu.sync_copy(ids_hbm.at[pl.ds(base, bpw)], ids_v)
        idvec = ids_v[...]
        blocks = (idvec >> 7) << 7
        lanes = idvec & (BLK - 1)
        iota = lax.iota(jnp.int32, LANES)

        def fire(i):
            blk = pl.multiple_of(blocks[i], BLK)
            return pltpu.async_copy(
                table_hbm.at[:, pl.ds(blk, BLK)], bufs[i % NBUF], sem
            )

        cps = [None] * bpw
        for i in range(NBUF):
            cps[i] = fire(i)
        pltpu.sync_copy(pe_hbm.at[pl.ds(base, bpw)], pe_v)
        for i in range(bpw):
            cps[i].wait()
            buf = bufs[i % NBUF]
            lane_b = jnp.broadcast_to(lanes[i], (LANES,))
            for j in range(EMBED_DIM // LANES):
                vals = plsc.load_gather(buf, [iota + (j * LANES), lane_b])
                sl = pl.ds(j * LANES, LANES)
                out_v[i, sl] = vals + pe_v[i, sl]
            if i + NBUF < bpw:
                cps[i + NBUF] = fire(i + NBUF)
        pltpu.sync_copy(out_v, out_hbm.at[pl.ds(base, bpw)])

    return emb_kernel(ids, table_t, pe)
